# baseline (speedup 1.0000x reference)
import functools

import jax
import jax.numpy as jnp
from jax import lax
from jax.experimental import pallas as pl
from jax.experimental.pallas import tpu as pltpu
from jax.experimental.pallas import tpu_sc as plsc

_BATCH = 16384
_SEQ = 200
_NUM_TOKENS = 1000
_LANES = 16
_COLS = 128
_UNROLL = 8


def _make_kernel():
  info = plsc.get_sparse_core_info()
  nc, ns = info.num_cores, info.num_subcores
  nw = nc * ns
  cols_per_w = _BATCH // nw
  n_chunks = cols_per_w // _COLS
  n_pairs = n_chunks // 2
  groups = _COLS // _LANES
  n_steps = _SEQ // _UNROLL

  mesh = plsc.VectorSubcoreMesh(core_axis_name="c", subcore_axis_name="s")

  @functools.partial(
      pl.kernel,
      mesh=mesh,
      compiler_params=pltpu.CompilerParams(
          needs_layout_passes=False, use_tc_tiling_on_sc=True),
      out_type=jax.ShapeDtypeStruct((_BATCH,), jnp.float32),
      scratch_types=[
          pltpu.VMEM((_NUM_TOKENS,), jnp.float32),
          pltpu.VMEM((_LANES,), jnp.float32),
          pltpu.VMEM((_SEQ, _COLS), jnp.int32),
          pltpu.VMEM((_SEQ, _COLS), jnp.int32),
          pltpu.VMEM((_SEQ, _COLS), jnp.int32),
          pltpu.VMEM((_SEQ, _COLS), jnp.int32),
          pltpu.VMEM((cols_per_w,), jnp.float32),
          pltpu.SemaphoreType.DMA,
          pltpu.SemaphoreType.DMA,
      ],
  )
  def sc_kernel(counts_hbm, terms_hbm, weights_hbm, bias_hbm, out_hbm,
                w_v, b_v, c_v0, t_v0, c_v1, t_v1, o_v, sem0, sem1):
    wid = lax.axis_index("s") * nc + lax.axis_index("c")
    col0 = wid * cols_per_w

    bufs = ((c_v0, t_v0, sem0), (c_v1, t_v1, sem1))

    def issue(chunk, which):
      c_v, t_v, sem = bufs[which]
      c = col0 + chunk * _COLS
      pltpu.make_async_copy(
          counts_hbm.at[:, pl.ds(c, _COLS)], c_v, sem).start()
      pltpu.make_async_copy(
          terms_hbm.at[:, pl.ds(c, _COLS)], t_v, sem).start()

    issue(0, 0)
    pltpu.sync_copy(weights_hbm, w_v)
    pltpu.sync_copy(bias_hbm, b_v)
    bias_vec = b_v[...]
    zero = jnp.zeros((_LANES,), jnp.float32)

    def drain(which):
      c_v, t_v, sem = bufs[which]
      pltpu.make_async_copy(
          counts_hbm.at[:, pl.ds(0, _COLS)], c_v, sem).wait()
      pltpu.make_async_copy(
          terms_hbm.at[:, pl.ds(0, _COLS)], t_v, sem).wait()

    def compute(chunk, which):
      c_ref, t_ref, _ = bufs[which]

      @plsc.parallel_loop(0, groups, unroll=2)
      def group_body(g):
        col = g * _LANES

        def s_body(k, accs):
          accs = list(accs)
          s0 = k * _UNROLL
          for j in range(_UNROLL):
            t = t_ref[s0 + j, pl.ds(col, _LANES)]
            c = c_ref[s0 + j, pl.ds(col, _LANES)]
            w = plsc.load_gather(w_v, [t])
            accs[j % 4] = accs[j % 4] + c.astype(jnp.float32) * w
          return tuple(accs)

        a0, a1, a2, a3 = lax.fori_loop(
            0, n_steps, s_body, (zero, zero, zero, zero))
        x = (a0 + a1) + (a2 + a3) + bias_vec
        o_v[pl.ds(chunk * _COLS + col, _LANES)] = 1.0 / (1.0 + jnp.exp(-x))

    def pair_body(i, _):
      issue(2 * i + 1, 1)
      drain(0)
      compute(2 * i, 0)

      @pl.when(i < n_pairs - 1)
      def _():
        issue(2 * i + 2, 0)

      drain(1)
      compute(2 * i + 1, 1)
      return ()

    lax.fori_loop(0, n_pairs, pair_body, ())
    pltpu.sync_copy(o_v, out_hbm.at[pl.ds(wid * cols_per_w, cols_per_w)])

  return sc_kernel


_sc_kernel = _make_kernel()


@jax.jit
def kernel(counts, terms, weights, bias):
  bias_vec = jnp.broadcast_to(bias, (_LANES,)).astype(jnp.float32)
  return _sc_kernel(counts.T, terms.T, weights, bias_vec)


# speedup vs baseline: 1.0006x; 1.0006x over previous
import functools

import jax
import jax.numpy as jnp
from jax import lax
from jax.experimental import pallas as pl
from jax.experimental.pallas import tpu as pltpu
from jax.experimental.pallas import tpu_sc as plsc

_BATCH = 16384
_SEQ = 200
_NUM_TOKENS = 1000
_LANES = 16
_COLS = 128
_UNROLL = 8


def _make_kernel():
  info = plsc.get_sparse_core_info()
  nc, ns = info.num_cores, info.num_subcores
  nw = nc * ns
  cols_per_w = _BATCH // nw
  n_chunks = cols_per_w // _COLS
  n_pairs = n_chunks // 2
  groups = _COLS // _LANES
  n_steps = _SEQ // _UNROLL

  mesh = plsc.VectorSubcoreMesh(core_axis_name="c", subcore_axis_name="s")

  @functools.partial(
      pl.kernel,
      mesh=mesh,
      compiler_params=pltpu.CompilerParams(
          needs_layout_passes=False, use_tc_tiling_on_sc=True),
      out_type=jax.ShapeDtypeStruct((_BATCH,), jnp.float32),
      scratch_types=[
          pltpu.VMEM((_NUM_TOKENS,), jnp.float32),
          pltpu.VMEM((_LANES,), jnp.float32),
          pltpu.VMEM((_SEQ, _COLS), jnp.int32),
          pltpu.VMEM((_SEQ, _COLS), jnp.int32),
          pltpu.VMEM((_SEQ, _COLS), jnp.int32),
          pltpu.VMEM((_SEQ, _COLS), jnp.int32),
          pltpu.VMEM((cols_per_w,), jnp.float32),
          pltpu.SemaphoreType.DMA,
          pltpu.SemaphoreType.DMA,
      ],
  )
  def sc_kernel(counts_hbm, terms_hbm, weights_hbm, bias_hbm, out_hbm,
                w_v, b_v, c_v0, t_v0, c_v1, t_v1, o_v, sem0, sem1):
    wid = lax.axis_index("s") * nc + lax.axis_index("c")
    col0 = wid * cols_per_w

    bufs = ((c_v0, t_v0, sem0), (c_v1, t_v1, sem1))

    def issue(chunk, which):
      c_v, t_v, sem = bufs[which]
      c = col0 + chunk * _COLS
      pltpu.make_async_copy(
          counts_hbm.at[:, pl.ds(c, _COLS)], c_v, sem).start()
      pltpu.make_async_copy(
          terms_hbm.at[:, pl.ds(c, _COLS)], t_v, sem).start()

    issue(0, 0)
    pltpu.sync_copy(weights_hbm, w_v)
    pltpu.sync_copy(bias_hbm, b_v)
    bias_vec = b_v[...]
    zero = jnp.zeros((_LANES,), jnp.float32)

    def drain(which):
      c_v, t_v, sem = bufs[which]
      pltpu.make_async_copy(
          counts_hbm.at[:, pl.ds(0, _COLS)], c_v, sem).wait()
      pltpu.make_async_copy(
          terms_hbm.at[:, pl.ds(0, _COLS)], t_v, sem).wait()

    def compute(chunk, which):
      c_ref, t_ref, _ = bufs[which]

      @plsc.parallel_loop(0, groups)
      def group_body(g):
        col = g * _LANES

        def s_body(k, accs):
          accs = list(accs)
          s0 = k * _UNROLL
          for j in range(_UNROLL):
            t = t_ref[s0 + j, pl.ds(col, _LANES)]
            c = c_ref[s0 + j, pl.ds(col, _LANES)]
            w = plsc.load_gather(w_v, [t])
            accs[j % 4] = accs[j % 4] + c.astype(jnp.float32) * w
          return tuple(accs)

        a0, a1, a2, a3 = lax.fori_loop(
            0, n_steps, s_body, (zero, zero, zero, zero))
        x = (a0 + a1) + (a2 + a3) + bias_vec
        o_v[pl.ds(chunk * _COLS + col, _LANES)] = 1.0 / (1.0 + jnp.exp(-x))

    def pair_body(i, _):
      issue(2 * i + 1, 1)
      drain(0)
      compute(2 * i, 0)

      @pl.when(i < n_pairs - 1)
      def _():
        issue(2 * i + 2, 0)

      drain(1)
      compute(2 * i + 1, 1)
      return ()

    lax.fori_loop(0, n_pairs, pair_body, ())
    pltpu.sync_copy(o_v, out_hbm.at[pl.ds(wid * cols_per_w, cols_per_w)])

  return sc_kernel


_sc_kernel = _make_kernel()


@jax.jit
def kernel(counts, terms, weights, bias):
  bias_vec = jnp.broadcast_to(bias, (_LANES,)).astype(jnp.float32)
  return _sc_kernel(counts.T, terms.T, weights, bias_vec)
